# Initial kernel scaffold; baseline (speedup 1.0000x reference)
import jax
import jax.numpy as jnp
from jax.experimental import pallas as pl


def kernel(x1, x2, edges, Wg1, bg1, Wg2, bg2, Wg3, bg3, Wd1, bd1, Wd2, bd2):
    raise NotImplementedError("write your pallas kernel here")


# speedup vs baseline: 162.9690x; 162.9690x over previous
import jax
import jax.numpy as jnp
from jax import lax
from jax.experimental import pallas as pl
from jax.experimental.pallas import tpu as pltpu
from jax.experimental.pallas import tpu_sc as plsc

NC = 2
NS = 16
LN = 256
CH = 4
CB = CH * LN


def _mesh():
  return plsc.VectorSubcoreMesh(core_axis_name="c", subcore_axis_name="s")


def _chunk_range(c, s, k_sc, k_t, k_rem):
  base = c * k_sc + s * k_t + jnp.minimum(s, k_rem)
  count = k_t + jnp.where(s < k_rem, 1, 0)
  return base, count


def _deg_pass(NP, U):
  k_sc = U // CH // NC
  k_t = k_sc // NS
  k_rem = k_sc - k_t * NS
  rpt = NP // NS

  def body(edges_ref, ones_ref, out_ref, deg_sp, stage, dbuf, ones_v, sem,
           semi):
    c = lax.axis_index("c")
    s = lax.axis_index("s")
    r0 = s * rpt
    pltpu.sync_copy(ones_ref.at[pl.ds(r0, rpt)], stage)
    pltpu.sync_copy(stage, deg_sp.at[pl.ds(r0, rpt)])
    pltpu.sync_copy(ones_ref.at[pl.ds(0, LN)], ones_v)
    plsc.subcore_barrier()

    dst_u = edges_ref.at[1]
    kb, n_chunks = _chunk_range(c, s, k_sc, k_t, k_rem)
    pltpu.sync_copy(dst_u.at[pl.ds(pl.multiple_of(kb * CB, CB), CB)],
                    dbuf.at[0])

    def chunk(i, _):
      p = lax.rem(i, 2)
      nxt = pl.multiple_of((kb + jnp.minimum(i + 1, n_chunks - 1)) * CB, CB)
      ds = [pltpu.async_copy(ones_v, deg_sp.at[dbuf.at[p, pl.ds(j * LN, LN)]],
                             sem, add=True)
            for j in range(CH)]
      pre = pltpu.async_copy(dst_u.at[pl.ds(nxt, CB)], dbuf.at[1 - p], semi)
      for d in ds:
        d.wait()
      pre.wait()
      return 0

    lax.fori_loop(0, n_chunks, chunk, 0)
    plsc.subcore_barrier()
    pltpu.sync_copy(deg_sp.at[pl.ds(r0, rpt)], stage)
    pltpu.sync_copy(stage, out_ref.at[c].at[pl.ds(r0, rpt)])

  return pl.kernel(
      body,
      out_type=jax.ShapeDtypeStruct((NC, NP), jnp.float32),
      mesh=_mesh(),
      compiler_params=pltpu.CompilerParams(use_tc_tiling_on_sc=False),
      scratch_types=[
          pltpu.VMEM_SHARED((NP,), jnp.float32),
          pltpu.VMEM((rpt,), jnp.float32),
          pltpu.VMEM((2, CB), jnp.int32),
          pltpu.VMEM((LN,), jnp.float32),
          pltpu.SemaphoreType.DMA,
          pltpu.SemaphoreType.DMA,
      ],
  )


def _prop1_pass(NP, U):
  k_sc = U // CH // NC
  k_t = k_sc // NS
  k_rem = k_sc - k_t * NS
  rpt = NP // NS

  def body(edges_ref, a1_ref, dinv_ref, acc_out, s_out,
           tab_sp, acc_sp, dinv_sp, s_sp, stage3, stage1,
           sbuf, dbuf, rows, dvals, sem, sem2, semi):
    c = lax.axis_index("c")
    s = lax.axis_index("s")
    r0 = s * rpt
    pltpu.sync_copy(a1_ref.at[pl.ds(r0, rpt)], stage3)
    pltpu.sync_copy(stage3, tab_sp.at[pl.ds(r0, rpt)])
    pltpu.sync_copy(stage3, acc_sp.at[pl.ds(r0, rpt)])
    pltpu.sync_copy(dinv_ref.at[pl.ds(r0, rpt)], stage1)
    pltpu.sync_copy(stage1, dinv_sp.at[pl.ds(r0, rpt)])
    pltpu.sync_copy(stage1, s_sp.at[pl.ds(r0, rpt)])
    plsc.subcore_barrier()

    src_u = edges_ref.at[0]
    dst_u = edges_ref.at[1]
    kb, n_chunks = _chunk_range(c, s, k_sc, k_t, k_rem)
    row0 = pl.multiple_of(kb * CB, CB)
    pltpu.sync_copy(src_u.at[pl.ds(row0, CB)], sbuf.at[0])
    pltpu.sync_copy(dst_u.at[pl.ds(row0, CB)], dbuf.at[0])

    def chunk(i, _):
      p = lax.rem(i, 2)
      nxt = pl.multiple_of((kb + jnp.minimum(i + 1, n_chunks - 1)) * CB, CB)
      gs = []
      for j in range(CH):
        gs.append(pltpu.async_copy(tab_sp.at[sbuf.at[p, pl.ds(j * LN, LN)]],
                                   rows.at[pl.ds(j * LN, LN)], sem))
        gs.append(pltpu.async_copy(dinv_sp.at[dbuf.at[p, pl.ds(j * LN, LN)]],
                                   dvals.at[pl.ds(j * LN, LN)], sem))
      pre = [pltpu.async_copy(src_u.at[pl.ds(nxt, CB)], sbuf.at[1 - p], semi),
             pltpu.async_copy(dst_u.at[pl.ds(nxt, CB)], dbuf.at[1 - p], semi)]
      ss = []
      for j in range(CH):
        gs[2 * j].wait()
        gs[2 * j + 1].wait()
        ss.append(pltpu.async_copy(rows.at[pl.ds(j * LN, LN)],
                                   acc_sp.at[dbuf.at[p, pl.ds(j * LN, LN)]],
                                   sem2, add=True))
        ss.append(pltpu.async_copy(dvals.at[pl.ds(j * LN, LN)],
                                   s_sp.at[sbuf.at[p, pl.ds(j * LN, LN)]],
                                   sem2, add=True))
      for d in ss:
        d.wait()
      for d in pre:
        d.wait()
      return 0

    lax.fori_loop(0, n_chunks, chunk, 0)
    plsc.subcore_barrier()
    pltpu.sync_copy(acc_sp.at[pl.ds(r0, rpt)], stage3)
    pltpu.sync_copy(stage3, acc_out.at[c].at[pl.ds(r0, rpt)])
    pltpu.sync_copy(s_sp.at[pl.ds(r0, rpt)], stage1)
    pltpu.sync_copy(stage1, s_out.at[c].at[pl.ds(r0, rpt)])

  return pl.kernel(
      body,
      out_type=(jax.ShapeDtypeStruct((NC, NP, 3), jnp.float32),
                jax.ShapeDtypeStruct((NC, NP), jnp.float32)),
      mesh=_mesh(),
      compiler_params=pltpu.CompilerParams(use_tc_tiling_on_sc=False),
      scratch_types=[
          pltpu.VMEM_SHARED((NP, 3), jnp.float32),
          pltpu.VMEM_SHARED((NP, 3), jnp.float32),
          pltpu.VMEM_SHARED((NP,), jnp.float32),
          pltpu.VMEM_SHARED((NP,), jnp.float32),
          pltpu.VMEM((rpt, 3), jnp.float32),
          pltpu.VMEM((rpt,), jnp.float32),
          pltpu.VMEM((2, CB), jnp.int32),
          pltpu.VMEM((2, CB), jnp.int32),
          pltpu.VMEM((CH * LN, 3), jnp.float32),
          pltpu.VMEM((CB,), jnp.float32),
          pltpu.SemaphoreType.DMA,
          pltpu.SemaphoreType.DMA,
          pltpu.SemaphoreType.DMA,
      ],
  )


def _prop2_pass(NP, U):
  k_sc = U // CH // NC
  k_t = k_sc // NS
  k_rem = k_sc - k_t * NS
  rpt = NP // NS

  def body(edges_ref, a2_ref, acc_out,
           tab_sp, acc_sp, stage5, sbuf, dbuf, rows, sem, sem2, semi):
    c = lax.axis_index("c")
    s = lax.axis_index("s")
    r0 = s * rpt
    pltpu.sync_copy(a2_ref.at[pl.ds(r0, rpt)], stage5)
    pltpu.sync_copy(stage5, tab_sp.at[pl.ds(r0, rpt)])
    pltpu.sync_copy(stage5, acc_sp.at[pl.ds(r0, rpt)])
    plsc.subcore_barrier()

    src_u = edges_ref.at[0]
    dst_u = edges_ref.at[1]
    kb, n_chunks = _chunk_range(c, s, k_sc, k_t, k_rem)
    row0 = pl.multiple_of(kb * CB, CB)
    pltpu.sync_copy(src_u.at[pl.ds(row0, CB)], sbuf.at[0])
    pltpu.sync_copy(dst_u.at[pl.ds(row0, CB)], dbuf.at[0])

    def chunk(i, _):
      p = lax.rem(i, 2)
      nxt = pl.multiple_of((kb + jnp.minimum(i + 1, n_chunks - 1)) * CB, CB)
      gs = [pltpu.async_copy(tab_sp.at[sbuf.at[p, pl.ds(j * LN, LN)]],
                             rows.at[pl.ds(j * LN, LN)], sem)
            for j in range(CH)]
      pre = [pltpu.async_copy(src_u.at[pl.ds(nxt, CB)], sbuf.at[1 - p], semi),
             pltpu.async_copy(dst_u.at[pl.ds(nxt, CB)], dbuf.at[1 - p], semi)]
      ss = []
      for j in range(CH):
        gs[j].wait()
        ss.append(pltpu.async_copy(rows.at[pl.ds(j * LN, LN)],
                                   acc_sp.at[dbuf.at[p, pl.ds(j * LN, LN)]],
                                   sem2, add=True))
      for d in ss:
        d.wait()
      for d in pre:
        d.wait()
      return 0

    lax.fori_loop(0, n_chunks, chunk, 0)
    plsc.subcore_barrier()
    pltpu.sync_copy(acc_sp.at[pl.ds(r0, rpt)], stage5)
    pltpu.sync_copy(stage5, acc_out.at[c].at[pl.ds(r0, rpt)])

  return pl.kernel(
      body,
      out_type=jax.ShapeDtypeStruct((NC, NP, 5), jnp.float32),
      mesh=_mesh(),
      compiler_params=pltpu.CompilerParams(use_tc_tiling_on_sc=False),
      scratch_types=[
          pltpu.VMEM_SHARED((NP, 5), jnp.float32),
          pltpu.VMEM_SHARED((NP, 5), jnp.float32),
          pltpu.VMEM((rpt, 5), jnp.float32),
          pltpu.VMEM((2, CB), jnp.int32),
          pltpu.VMEM((2, CB), jnp.int32),
          pltpu.VMEM((CH * LN, 5), jnp.float32),
          pltpu.SemaphoreType.DMA,
          pltpu.SemaphoreType.DMA,
          pltpu.SemaphoreType.DMA,
      ],
  )


def _glue_a(NP, R):
  grid = NP // R

  def body(deg_ref, x1_ref, dinv_ref, a1_ref):
    deg = deg_ref[0, :] + deg_ref[1, :] - 1.0
    dinv = lax.rsqrt(deg)
    dinv_ref[...] = dinv[:, None]
    a1_ref[...] = x1_ref[...] * dinv[:, None]

  return pl.pallas_call(
      body,
      grid=(grid,),
      in_specs=[
          pl.BlockSpec((NC, R), lambda i: (0, i)),
          pl.BlockSpec((R, 3), lambda i: (i, 0)),
      ],
      out_specs=[
          pl.BlockSpec((R, 1), lambda i: (i, 0)),
          pl.BlockSpec((R, 3), lambda i: (i, 0)),
      ],
      out_shape=[
          jax.ShapeDtypeStruct((NP, 1), jnp.float32),
          jax.ShapeDtypeStruct((NP, 3), jnp.float32),
      ],
  )


def _glue_b(NP, R):
  grid = NP // R

  def body(acc_ref, a1_ref, dinv_ref, w1_ref, b1_ref, a2_ref):
    p = (acc_ref[0] + acc_ref[1] - a1_ref[...]) * dinv_ref[...]
    h = jnp.dot(p, w1_ref[...], preferred_element_type=jnp.float32)
    h = jnp.maximum(h + b1_ref[...], 0.0)
    a2_ref[...] = h * dinv_ref[...]

  return pl.pallas_call(
      body,
      grid=(grid,),
      in_specs=[
          pl.BlockSpec((NC, R, 3), lambda i: (0, i, 0)),
          pl.BlockSpec((R, 3), lambda i: (i, 0)),
          pl.BlockSpec((R, 1), lambda i: (i, 0)),
          pl.BlockSpec((3, 5), lambda i: (0, 0)),
          pl.BlockSpec((1, 5), lambda i: (0, 0)),
      ],
      out_specs=pl.BlockSpec((R, 5), lambda i: (i, 0)),
      out_shape=jax.ShapeDtypeStruct((NP, 5), jnp.float32),
  )


def _glue_c(NP, R, N):
  grid = NP // R

  def body(acc_ref, a2_ref, dinv_ref, s_ref, w2_ref, b2_ref, w3_ref, b3_ref,
           x2_ref, wd1_ref, bd1_ref, wd2_ref, bd2_ref, out_ref, uacc):
    i = pl.program_id(0)
    p = (acc_ref[0] + acc_ref[1] - a2_ref[...]) * dinv_ref[...]
    h2 = jnp.dot(p, w2_ref[...], preferred_element_type=jnp.float32)
    h2 = jnp.maximum(h2 + b2_ref[...], 0.0)
    svec = (s_ref[0, :] + s_ref[1, :])[:, None] - dinv_ref[...]
    cvec = dinv_ref[...] * svec
    row = lax.broadcasted_iota(jnp.int32, (R, 1), 0) + i * R
    cvec = jnp.where(row < N, cvec, 0.0)
    u = lax.dot_general(cvec, h2, (((0,), (0,)), ((), ())),
                        preferred_element_type=jnp.float32)

    @pl.when(i == 0)
    def _():
      uacc[...] = jnp.zeros_like(uacc)

    uacc[...] += u

    @pl.when(i == grid - 1)
    def _():
      m = jnp.dot(uacc[...], w3_ref[...],
                  preferred_element_type=jnp.float32) / N + b3_ref[...]
      v = jnp.concatenate([m, x2_ref[...]], axis=1)
      v = jnp.maximum(jnp.dot(v, wd1_ref[...],
                              preferred_element_type=jnp.float32)
                      + bd1_ref[...], 0.0)
      o = jnp.dot(v, wd2_ref[...], preferred_element_type=jnp.float32)
      out_ref[...] = jnp.tanh(o + bd2_ref[...])

  full = lambda i: (0, 0)
  return pl.pallas_call(
      body,
      grid=(grid,),
      in_specs=[
          pl.BlockSpec((NC, R, 5), lambda i: (0, i, 0)),
          pl.BlockSpec((R, 5), lambda i: (i, 0)),
          pl.BlockSpec((R, 1), lambda i: (i, 0)),
          pl.BlockSpec((NC, R), lambda i: (0, i)),
          pl.BlockSpec((5, 5), full),
          pl.BlockSpec((1, 5), full),
          pl.BlockSpec((5, 1), full),
          pl.BlockSpec((1, 1), full),
          pl.BlockSpec((1, 4), full),
          pl.BlockSpec((5, 15), full),
          pl.BlockSpec((1, 15), full),
          pl.BlockSpec((15, 1), full),
          pl.BlockSpec((1, 1), full),
      ],
      out_specs=pl.BlockSpec((1, 1), full),
      out_shape=jax.ShapeDtypeStruct((1, 1), jnp.float32),
      scratch_shapes=[pltpu.VMEM((1, 5), jnp.float32)],
  )


def kernel(x1, x2, edges, Wg1, bg1, Wg2, bg2, Wg3, bg3, Wd1, bd1, Wd2, bd2):
  N = x1.shape[0]
  E = edges.shape[1]
  NP = 100352
  R = 6272
  U = E // LN

  x1p = jnp.pad(x1, ((0, NP - N), (0, 0)))
  ones_np = jnp.ones((NP,), jnp.float32)

  deg = _deg_pass(NP, U)(edges, ones_np)
  dinv, a1 = _glue_a(NP, R)(deg, x1p)
  dinv1 = dinv.reshape(NP)

  acc1, svec = _prop1_pass(NP, U)(edges, a1, dinv1)
  a2 = _glue_b(NP, R)(acc1, a1, dinv, Wg1, bg1.reshape(1, 5))

  acc2 = _prop2_pass(NP, U)(edges, a2)
  out = _glue_c(NP, R, N)(acc2, a2, dinv, svec, Wg2, bg2.reshape(1, 5),
                          Wg3, bg3.reshape(1, 1), x2.reshape(1, 4), Wd1,
                          bd1.reshape(1, 15), Wd2, bd2.reshape(1, 1))
  return out[0, 0]
